# Initial kernel scaffold; baseline (speedup 1.0000x reference)
import functools

import jax
import jax.numpy as jnp
from jax import lax
from jax.experimental import pallas as pl
from jax.experimental.pallas import tpu as pltpu
from jax.experimental.pallas import tpu_sc as plsc

NC = 2
NS = 16
NW = NC * NS
CH = 128


def _dense_body(h_ref, wg_ref, wa_ref, ba_ref, wb_ref, bb_ref, wc_ref,
                bc_ref, g_ref, a_ref, c_ref):
    h = h_ref[...]
    g_ref[...] = jnp.dot(h, wg_ref[...], preferred_element_type=jnp.float32)
    a = jnp.dot(h, wa_ref[...], preferred_element_type=jnp.float32) + ba_ref[...]
    a_ref[...] = jnp.maximum(a, 0.0)
    tb = jnp.dot(h, wb_ref[...], preferred_element_type=jnp.float32) + bb_ref[...]
    tc = jnp.dot(h, wc_ref[...], preferred_element_type=jnp.float32) + bc_ref[...]
    c_ref[...] = jnp.maximum(tb * tc, 0.0)


def _dense(h, wg, wa, ba, wb, bb, wc, bc, bn):
    n, fan = h.shape
    nout = wg.shape[1]
    grid = n // bn
    full = lambda i: (0, 0)
    row = lambda i: (i, 0)
    return pl.pallas_call(
        _dense_body,
        grid=(grid,),
        in_specs=[
            pl.BlockSpec((bn, fan), row),
            pl.BlockSpec((fan, nout), full),
            pl.BlockSpec((fan, nout), full),
            pl.BlockSpec((1, nout), full),
            pl.BlockSpec((fan, nout), full),
            pl.BlockSpec((1, nout), full),
            pl.BlockSpec((fan, nout), full),
            pl.BlockSpec((1, nout), full),
        ],
        out_specs=[
            pl.BlockSpec((bn, nout), row),
            pl.BlockSpec((bn, nout), row),
            pl.BlockSpec((bn, nout), row),
        ],
        out_shape=[jax.ShapeDtypeStruct((n, nout), jnp.float32)] * 3,
    )(h, wg, wa, ba.reshape(1, -1), wb, bb.reshape(1, -1), wc,
      bc.reshape(1, -1))


def _assemble_body(a_ref, agg_ref, c_ref, bias_ref, h_ref):
    agg = agg_ref[...]
    b_ = jnp.maximum(agg[0] + agg[1] + bias_ref[...], 0.0)
    h_ref[...] = jnp.concatenate([a_ref[...], b_, c_ref[...]], axis=1)


def _assemble(a, agg, c, bias, bn):
    n, nout = a.shape
    grid = n // bn
    return pl.pallas_call(
        _assemble_body,
        grid=(grid,),
        in_specs=[
            pl.BlockSpec((bn, nout), lambda i: (i, 0)),
            pl.BlockSpec((2, bn, nout), lambda i: (0, i, 0)),
            pl.BlockSpec((bn, nout), lambda i: (i, 0)),
            pl.BlockSpec((1, nout), lambda i: (0, 0)),
        ],
        out_specs=pl.BlockSpec((bn, 3 * nout), lambda i: (i, 0)),
        out_shape=jax.ShapeDtypeStruct((n, 3 * nout), jnp.float32),
    )(a, agg, c, bias.reshape(1, -1))


def _pool_body(a_ref, agg_ref, c_ref, bias_ref, batch_ref, w1_ref, b1_ref,
               w2_ref, b2_ref, out_ref, acc_ref):
    i = pl.program_id(0)
    agg = agg_ref[...]
    b_ = jnp.maximum(agg[0] + agg[1] + bias_ref[...], 0.0)
    h = jnp.concatenate([a_ref[...], b_, c_ref[...]], axis=1)
    gid = batch_ref[...]
    ng = acc_ref.shape[0]
    onehot = (gid == lax.broadcasted_iota(jnp.int32, (1, ng), 1)
              ).astype(jnp.float32)
    part = lax.dot_general(onehot, h, (((0,), (0,)), ((), ())),
                           preferred_element_type=jnp.float32)

    @pl.when(i == 0)
    def _():
        acc_ref[...] = jnp.zeros_like(acc_ref)

    acc_ref[...] += part

    @pl.when(i == pl.num_programs(0) - 1)
    def _():
        o = jnp.dot(acc_ref[...], w1_ref[...],
                    preferred_element_type=jnp.float32) + b1_ref[...]
        o = jnp.dot(o, w2_ref[...],
                    preferred_element_type=jnp.float32) + b2_ref[...]
        out_ref[...] = o


def _pool(a, agg, c, bias, batch2, w1, b1, w2, b2, ng, bn):
    n, nout = a.shape
    grid = n // bn
    nin = 3 * nout
    nh = w1.shape[1]
    return pl.pallas_call(
        _pool_body,
        grid=(grid,),
        in_specs=[
            pl.BlockSpec((bn, nout), lambda i: (i, 0)),
            pl.BlockSpec((2, bn, nout), lambda i: (0, i, 0)),
            pl.BlockSpec((bn, nout), lambda i: (i, 0)),
            pl.BlockSpec((1, nout), lambda i: (0, 0)),
            pl.BlockSpec((bn, 1), lambda i: (i, 0)),
            pl.BlockSpec((nin, nh), lambda i: (0, 0)),
            pl.BlockSpec((1, nh), lambda i: (0, 0)),
            pl.BlockSpec((nh, 1), lambda i: (0, 0)),
            pl.BlockSpec((1, 1), lambda i: (0, 0)),
        ],
        out_specs=pl.BlockSpec((ng, 1), lambda i: (0, 0)),
        out_shape=jax.ShapeDtypeStruct((ng, 1), jnp.float32),
        scratch_shapes=[pltpu.VMEM((ng, nin), jnp.float32)],
    )(a, agg, c, bias.reshape(1, -1), batch2, w1, b1.reshape(1, -1), w2,
      b2.reshape(1, -1))


def _make_scatter(n_pad, n_chunks, nout):
    rows_per = n_pad // NS
    mesh = plsc.VectorSubcoreMesh(core_axis_name="c", subcore_axis_name="s")

    @functools.partial(
        pl.kernel, mesh=mesh,
        out_type=jax.ShapeDtypeStruct((NC, n_pad, nout), jnp.float32),
        scratch_types=[
            pltpu.VMEM((n_chunks, CH), jnp.int32),
            pltpu.VMEM((n_chunks, CH), jnp.int32),
            pltpu.VMEM((CH, nout), jnp.float32),
            pltpu.VMEM_SHARED((n_pad, nout), jnp.float32),
            pltpu.SemaphoreType.DMA,
        ],
    )
    def scatter(g_hbm, src_hbm, dst_hbm, zeros_hbm, out_hbm,
                src_v, dst_v, gbuf, acc, sem):
        c = lax.axis_index("c")
        s = lax.axis_index("s")
        wid = s * NC + c
        pltpu.sync_copy(zeros_hbm, acc.at[pl.ds(s * rows_per, rows_per)])
        pltpu.sync_copy(src_hbm.at[wid], src_v)
        pltpu.sync_copy(dst_hbm.at[wid], dst_v)
        plsc.subcore_barrier()

        def body(j, carry):
            pltpu.async_copy(g_hbm.at[src_v.at[j]], gbuf, sem).wait()
            pltpu.sync_copy(gbuf, acc.at[dst_v.at[j]], add=True)
            return carry

        lax.fori_loop(0, n_chunks, body, 0)
        plsc.subcore_barrier()
        pltpu.sync_copy(acc.at[pl.ds(s * rows_per, rows_per)],
                        out_hbm.at[c, pl.ds(s * rows_per, rows_per)])

    return scatter


def kernel(x, edge_index, batch, params):
    n, d = x.shape
    e = edge_index.shape[1]
    nout = params['conv0_W'].shape[1]
    ng = 64
    bn = 1000

    n_chunks = -(-e // (NW * CH))
    e_pad = NW * n_chunks * CH
    n_pad = -(-(n + 1) // NS) * NS

    src = edge_index[0]
    dst = edge_index[1]
    srcp = jnp.concatenate([src, jnp.zeros((e_pad - e,), jnp.int32)])
    dstp = jnp.concatenate([dst, jnp.full((e_pad - e,), n, jnp.int32)])
    srcr = srcp.reshape(NW, n_chunks, CH)
    dstr = dstp.reshape(NW, n_chunks, CH)
    zeros = jnp.zeros((n_pad // NS, nout), jnp.float32)
    batch2 = batch.reshape(n, 1)

    scatter = _make_scatter(n_pad, n_chunks, nout)

    h = x
    out = None
    for i in range(5):
        g, a, c = _dense(h, params[f'conv{i}_W'],
                         params[f'fc_a{i}_W'], params[f'fc_a{i}_b'],
                         params[f'fc_b{i}_W'], params[f'fc_b{i}_b'],
                         params[f'fc_c{i}_W'], params[f'fc_c{i}_b'], bn)
        agg = scatter(g, srcr, dstr, zeros)
        if i < 4:
            h = _assemble(a, agg, c, params[f'conv{i}_b'], bn)
        else:
            out = _pool(a, agg, c, params[f'conv{i}_b'], batch2,
                        params['fc1_W'], params['fc1_b'],
                        params['fc2_W'], params['fc2_b'], ng, bn)
    return out


# speedup vs baseline: 9.3229x; 9.3229x over previous
import functools

import jax
import jax.numpy as jnp
from jax import lax
from jax.experimental import pallas as pl
from jax.experimental.pallas import tpu as pltpu
from jax.experimental.pallas import tpu_sc as plsc

NC = 2
NS = 16
NW = NC * NS
CH = 128


def _dense_body(h_ref, wg_ref, wa_ref, ba_ref, wb_ref, bb_ref, wc_ref,
                bc_ref, g_ref, a_ref, c_ref):
    h = h_ref[...]
    g_ref[...] = jnp.dot(h, wg_ref[...], preferred_element_type=jnp.float32)
    a = jnp.dot(h, wa_ref[...], preferred_element_type=jnp.float32) + ba_ref[...]
    a_ref[...] = jnp.maximum(a, 0.0)
    tb = jnp.dot(h, wb_ref[...], preferred_element_type=jnp.float32) + bb_ref[...]
    tc = jnp.dot(h, wc_ref[...], preferred_element_type=jnp.float32) + bc_ref[...]
    c_ref[...] = jnp.maximum(tb * tc, 0.0)


def _dense(h, wg, wa, ba, wb, bb, wc, bc, bn):
    n, fan = h.shape
    nout = wg.shape[1]
    grid = n // bn
    full = lambda i: (0, 0)
    row = lambda i: (i, 0)
    return pl.pallas_call(
        _dense_body,
        grid=(grid,),
        in_specs=[
            pl.BlockSpec((bn, fan), row),
            pl.BlockSpec((fan, nout), full),
            pl.BlockSpec((fan, nout), full),
            pl.BlockSpec((1, nout), full),
            pl.BlockSpec((fan, nout), full),
            pl.BlockSpec((1, nout), full),
            pl.BlockSpec((fan, nout), full),
            pl.BlockSpec((1, nout), full),
        ],
        out_specs=[
            pl.BlockSpec((bn, nout), row),
            pl.BlockSpec((bn, nout), row),
            pl.BlockSpec((bn, nout), row),
        ],
        out_shape=[jax.ShapeDtypeStruct((n, nout), jnp.float32)] * 3,
    )(h, wg, wa, ba.reshape(1, -1), wb, bb.reshape(1, -1), wc,
      bc.reshape(1, -1))


def _assemble_body(a_ref, agg_ref, c_ref, bias_ref, h_ref):
    agg = agg_ref[...]
    b_ = jnp.maximum(agg[0] + agg[1] + bias_ref[...], 0.0)
    h_ref[...] = jnp.concatenate([a_ref[...], b_, c_ref[...]], axis=1)


def _assemble(a, agg, c, bias, bn):
    n, nout = a.shape
    grid = n // bn
    return pl.pallas_call(
        _assemble_body,
        grid=(grid,),
        in_specs=[
            pl.BlockSpec((bn, nout), lambda i: (i, 0)),
            pl.BlockSpec((2, bn, nout), lambda i: (0, i, 0)),
            pl.BlockSpec((bn, nout), lambda i: (i, 0)),
            pl.BlockSpec((1, nout), lambda i: (0, 0)),
        ],
        out_specs=pl.BlockSpec((bn, 3 * nout), lambda i: (i, 0)),
        out_shape=jax.ShapeDtypeStruct((n, 3 * nout), jnp.float32),
    )(a, agg, c, bias.reshape(1, -1))


def _pool_body(a_ref, agg_ref, c_ref, bias_ref, batch_ref, w1_ref, b1_ref,
               w2_ref, b2_ref, out_ref, acc_ref):
    i = pl.program_id(0)
    agg = agg_ref[...]
    b_ = jnp.maximum(agg[0] + agg[1] + bias_ref[...], 0.0)
    h = jnp.concatenate([a_ref[...], b_, c_ref[...]], axis=1)
    gid = batch_ref[...]
    ng = acc_ref.shape[0]
    onehot = (gid == lax.broadcasted_iota(jnp.int32, (1, ng), 1)
              ).astype(jnp.float32)
    part = lax.dot_general(onehot, h, (((0,), (0,)), ((), ())),
                           preferred_element_type=jnp.float32)

    @pl.when(i == 0)
    def _():
        acc_ref[...] = jnp.zeros_like(acc_ref)

    acc_ref[...] += part

    @pl.when(i == pl.num_programs(0) - 1)
    def _():
        o = jnp.dot(acc_ref[...], w1_ref[...],
                    preferred_element_type=jnp.float32) + b1_ref[...]
        o = jnp.dot(o, w2_ref[...],
                    preferred_element_type=jnp.float32) + b2_ref[...]
        out_ref[...] = o


def _pool(a, agg, c, bias, batch2, w1, b1, w2, b2, ng, bn):
    n, nout = a.shape
    grid = n // bn
    nin = 3 * nout
    nh = w1.shape[1]
    return pl.pallas_call(
        _pool_body,
        grid=(grid,),
        in_specs=[
            pl.BlockSpec((bn, nout), lambda i: (i, 0)),
            pl.BlockSpec((2, bn, nout), lambda i: (0, i, 0)),
            pl.BlockSpec((bn, nout), lambda i: (i, 0)),
            pl.BlockSpec((1, nout), lambda i: (0, 0)),
            pl.BlockSpec((bn, 1), lambda i: (i, 0)),
            pl.BlockSpec((nin, nh), lambda i: (0, 0)),
            pl.BlockSpec((1, nh), lambda i: (0, 0)),
            pl.BlockSpec((nh, 1), lambda i: (0, 0)),
            pl.BlockSpec((1, 1), lambda i: (0, 0)),
        ],
        out_specs=pl.BlockSpec((ng, 1), lambda i: (0, 0)),
        out_shape=jax.ShapeDtypeStruct((ng, 1), jnp.float32),
        scratch_shapes=[pltpu.VMEM((ng, nin), jnp.float32)],
    )(a, agg, c, bias.reshape(1, -1), batch2, w1, b1.reshape(1, -1), w2,
      b2.reshape(1, -1))


def _make_scatter(n_pad, n_chunks, nout):
    rows_per = n_pad // NS
    mesh = plsc.VectorSubcoreMesh(core_axis_name="c", subcore_axis_name="s")

    @functools.partial(
        pl.kernel, mesh=mesh,
        compiler_params=pltpu.CompilerParams(use_tc_tiling_on_sc=False),
        out_type=jax.ShapeDtypeStruct((NC, n_pad, nout), jnp.float32),
        scratch_types=[
            pltpu.VMEM((n_chunks, CH), jnp.int32),
            pltpu.VMEM((n_chunks, CH), jnp.int32),
            pltpu.VMEM((CH, nout), jnp.float32),
            pltpu.VMEM_SHARED((n_pad, nout), jnp.float32),
            pltpu.SemaphoreType.DMA,
        ],
    )
    def scatter(g_hbm, src_hbm, dst_hbm, zeros_hbm, out_hbm,
                src_v, dst_v, gbuf, acc, sem):
        c = lax.axis_index("c")
        s = lax.axis_index("s")
        wid = s * NC + c
        pltpu.sync_copy(zeros_hbm, acc.at[pl.ds(s * rows_per, rows_per)])
        pltpu.sync_copy(src_hbm.at[wid], src_v)
        pltpu.sync_copy(dst_hbm.at[wid], dst_v)
        plsc.subcore_barrier()

        def body(j, carry):
            pltpu.async_copy(g_hbm.at[src_v.at[j]], gbuf, sem).wait()
            pltpu.sync_copy(gbuf, acc.at[dst_v.at[j]], add=True)
            return carry

        lax.fori_loop(0, n_chunks, body, 0)
        plsc.subcore_barrier()
        pltpu.sync_copy(acc.at[pl.ds(s * rows_per, rows_per)],
                        out_hbm.at[c, pl.ds(s * rows_per, rows_per)])

    return scatter


def kernel(x, edge_index, batch, params):
    n, d = x.shape
    e = edge_index.shape[1]
    nout = params['conv0_W'].shape[1]
    ng = 64
    bn = 1000

    n_chunks = -(-e // (NW * CH))
    e_pad = NW * n_chunks * CH
    n_pad = -(-(n + 1) // (NS * 8)) * (NS * 8)

    src = edge_index[0]
    dst = edge_index[1]
    srcp = jnp.concatenate([src, jnp.zeros((e_pad - e,), jnp.int32)])
    dstp = jnp.concatenate([dst, jnp.full((e_pad - e,), n, jnp.int32)])
    srcr = srcp.reshape(NW, n_chunks, CH)
    dstr = dstp.reshape(NW, n_chunks, CH)
    zeros = jnp.zeros((n_pad // NS, nout), jnp.float32)
    batch2 = batch.reshape(n, 1)

    scatter = _make_scatter(n_pad, n_chunks, nout)

    h = x
    out = None
    for i in range(5):
        g, a, c = _dense(h, params[f'conv{i}_W'],
                         params[f'fc_a{i}_W'], params[f'fc_a{i}_b'],
                         params[f'fc_b{i}_W'], params[f'fc_b{i}_b'],
                         params[f'fc_c{i}_W'], params[f'fc_c{i}_b'], bn)
        agg = scatter(g, srcr, dstr, zeros)
        if i < 4:
            h = _assemble(a, agg, c, params[f'conv{i}_b'], bn)
        else:
            out = _pool(a, agg, c, params[f'conv{i}_b'], batch2,
                        params['fc1_W'], params['fc1_b'],
                        params['fc2_W'], params['fc2_b'], ng, bn)
    return out
